# baseline (speedup 1.0000x reference)
import functools

import jax
import jax.numpy as jnp
from jax import lax
from jax.experimental import pallas as pl
from jax.experimental.pallas import tpu as pltpu
from jax.experimental.pallas import tpu_sc as plsc

_B = 16384
_D = 64
_NW = 32
_BPW = _B // _NW
_CH = 128
_NCH = _BPW // _CH
_W = 8
_L = 16

_mesh = plsc.VectorSubcoreMesh(core_axis_name="c", subcore_axis_name="s")


@functools.partial(
    pl.kernel,
    mesh=_mesh,
    compiler_params=pltpu.CompilerParams(
        use_tc_tiling_on_sc=False, needs_layout_passes=False),
    out_type=[
        jax.ShapeDtypeStruct((_NW * _NCH, _CH, _D), jnp.float32),
        jax.ShapeDtypeStruct((_NW * _NCH, _CH), jnp.float32),
        jax.ShapeDtypeStruct((_NW * _NCH, _CH), jnp.float32),
    ],
    scratch_types=[
        pltpu.VMEM((_NCH, _CH), jnp.int32),
        pltpu.VMEM((_NCH, _CH), jnp.int32),
        pltpu.VMEM((_NCH, _CH, _D), jnp.float32),
        pltpu.VMEM((_NCH, _CH, _W), jnp.float32),
        pltpu.VMEM((_NCH, _CH, _W), jnp.float32),
        pltpu.VMEM((_NCH, _CH), jnp.float32),
        pltpu.VMEM((_NCH, _CH), jnp.float32),
        pltpu.SemaphoreType.DMA,
        pltpu.SemaphoreType.DMA,
    ],
)
def _gather_all(idx_hbm, hi_hbm, emb_hbm, inh_hbm, det_hbm,
                z_out, inh_out, det_out,
                idx_v, hi_v, rows_v, inh_rows, det_rows,
                inh_sel, det_sel, sem, sem2):
    wid = lax.axis_index("s") * 2 + lax.axis_index("c")
    base = wid * _NCH
    pltpu.sync_copy(idx_hbm.at[pl.ds(base, _NCH)], idx_v)
    pltpu.sync_copy(hi_hbm.at[pl.ds(base, _NCH)], hi_v)
    copies = []
    for j in range(_NCH):
        copies.append(pltpu.async_copy(emb_hbm.at[idx_v.at[j]], rows_v.at[j], sem))
        copies.append(pltpu.async_copy(inh_hbm.at[hi_v.at[j]], inh_rows.at[j], sem))
        copies.append(pltpu.async_copy(det_hbm.at[hi_v.at[j]], det_rows.at[j], sem))
    for c in copies:
        c.wait()
    zcopy = pltpu.async_copy(rows_v, z_out.at[pl.ds(base, _NCH)], sem2)
    iotas = [lax.iota(jnp.int32, _L) + (_L * t) for t in range(_CH // _L)]
    for j in range(_NCH):
        for t in range(_CH // _L):
            o = _L * t
            lo = lax.bitwise_and(idx_v[j, pl.ds(o, _L)], 7)
            row = iotas[t]
            inh_sel[j, pl.ds(o, _L)] = plsc.load_gather(
                inh_rows.at[j], [row, lo])
            det_sel[j, pl.ds(o, _L)] = plsc.load_gather(
                det_rows.at[j], [row, lo])
    pltpu.sync_copy(inh_sel, inh_out.at[pl.ds(base, _NCH)])
    pltpu.sync_copy(det_sel, det_out.at[pl.ds(base, _NCH)])
    zcopy.wait()


def kernel(piano_model, embedding, inharm_embedding, detune_embedding):
    idx0 = piano_model.astype(jnp.int32)
    perm = jnp.argsort(idx0)
    sidx = idx0[perm]
    rank = jnp.zeros((_B,), jnp.int32).at[perm].set(
        jnp.arange(_B, dtype=jnp.int32))
    idx = sidx[rank]
    idx2d = idx.reshape(_NW * _NCH, _CH)
    hi2d = (idx >> 3).reshape(_NW * _NCH, _CH)
    z, inh, det = _gather_all(
        idx2d, hi2d, embedding,
        inharm_embedding.reshape(-1, _W),
        detune_embedding.reshape(-1, _W))
    return (z.reshape(_B, 1, _D),
            inh.reshape(_B, 1, 1),
            det.reshape(_B, 1, 1))


# speedup vs baseline: 2.4263x; 2.4263x over previous
import functools

import jax
import jax.numpy as jnp
from jax import lax
from jax.experimental import pallas as pl
from jax.experimental.pallas import tpu as pltpu
from jax.experimental.pallas import tpu_sc as plsc

_B = 16384
_D = 64
_NW = 32
_BPW = _B // _NW
_CH = 128
_NCH = _BPW // _CH
_W = 8
_L = 16

_mesh = plsc.VectorSubcoreMesh(core_axis_name="c", subcore_axis_name="s")


@functools.partial(
    pl.kernel,
    mesh=_mesh,
    compiler_params=pltpu.CompilerParams(
        use_tc_tiling_on_sc=True, needs_layout_passes=False),
    out_type=jax.ShapeDtypeStruct((_B, _D), jnp.float32),
    scratch_types=[
        pltpu.VMEM((_BPW, _D), jnp.float32),
        pltpu.VMEM((_BPW,), jnp.int32),
        pltpu.SemaphoreType.DMA,
    ],
)
def _gather_z(idx_hbm, emb_hbm, z_out, out_v, idx_v, sem):
    wid = lax.axis_index("s") * 2 + lax.axis_index("c")
    pltpu.sync_copy(idx_hbm.at[pl.ds(wid * _BPW, _BPW)], idx_v)

    def body(g, carry):
        vec = idx_v[pl.ds(g * _L, _L)]
        for k in range(_L):
            pltpu.async_copy(
                emb_hbm.at[vec[k]], out_v.at[g * _L + k], sem)
        @pl.when(g >= 2)
        def _():
            pltpu.make_async_copy(
                emb_hbm.at[pl.ds(0, _L)], out_v.at[pl.ds(0, _L)], sem).wait()
        return carry

    lax.fori_loop(0, _BPW // _L, body, 0)
    for _ in range(2):
        pltpu.make_async_copy(
            emb_hbm.at[pl.ds(0, _L)], out_v.at[pl.ds(0, _L)], sem).wait()
    pltpu.sync_copy(out_v, z_out.at[pl.ds(wid * _BPW, _BPW)])


@functools.partial(
    pl.kernel,
    mesh=_mesh,
    compiler_params=pltpu.CompilerParams(
        use_tc_tiling_on_sc=False, needs_layout_passes=False),
    out_type=[
        jax.ShapeDtypeStruct((_NW * _NCH, _CH), jnp.float32),
        jax.ShapeDtypeStruct((_NW * _NCH, _CH), jnp.float32),
    ],
    scratch_types=[
        pltpu.VMEM((_NCH, _CH), jnp.int32),
        pltpu.VMEM((_NCH, _CH), jnp.int32),
        pltpu.VMEM((_NCH, _CH, _W), jnp.float32),
        pltpu.VMEM((_NCH, _CH, _W), jnp.float32),
        pltpu.VMEM((_NCH, _CH), jnp.float32),
        pltpu.VMEM((_NCH, _CH), jnp.float32),
        pltpu.SemaphoreType.DMA,
    ],
)
def _gather_small(idx_hbm, hi3_hbm, inh_hbm, det_hbm,
                  inh_out, det_out,
                  idx_v, hi3_v, inh_rows, det_rows,
                  inh_sel, det_sel, sem):
    wid = lax.axis_index("s") * 2 + lax.axis_index("c")
    base = wid * _NCH
    pltpu.sync_copy(idx_hbm.at[pl.ds(base, _NCH)], idx_v)
    pltpu.sync_copy(hi3_hbm.at[pl.ds(base, _NCH)], hi3_v)
    copies = []
    for j in range(_NCH):
        copies.append(pltpu.async_copy(inh_hbm.at[hi3_v.at[j]], inh_rows.at[j], sem))
        copies.append(pltpu.async_copy(det_hbm.at[hi3_v.at[j]], det_rows.at[j], sem))
    for c in copies:
        c.wait()
    iotas = [lax.iota(jnp.int32, _L) + (_L * t) for t in range(_CH // _L)]
    for j in range(_NCH):
        for t in range(_CH // _L):
            o = _L * t
            lo = lax.bitwise_and(idx_v[j, pl.ds(o, _L)], 7)
            row = iotas[t]
            inh_sel[j, pl.ds(o, _L)] = plsc.load_gather(
                inh_rows.at[j], [row, lo])
            det_sel[j, pl.ds(o, _L)] = plsc.load_gather(
                det_rows.at[j], [row, lo])
    pltpu.sync_copy(inh_sel, inh_out.at[pl.ds(base, _NCH)])
    pltpu.sync_copy(det_sel, det_out.at[pl.ds(base, _NCH)])


def kernel(piano_model, embedding, inharm_embedding, detune_embedding):
    idx = piano_model.astype(jnp.int32)
    idx2d = idx.reshape(_NW * _NCH, _CH)
    hi3_2d = (idx >> 3).reshape(_NW * _NCH, _CH)
    z2 = _gather_z(idx, embedding)
    inh, det = _gather_small(
        idx2d, hi3_2d,
        inharm_embedding.reshape(-1, _W),
        detune_embedding.reshape(-1, _W))
    return (z2.reshape(_B, 1, _D),
            inh.reshape(_B, 1, 1),
            det.reshape(_B, 1, 1))


# speedup vs baseline: 2.5415x; 1.0475x over previous
import functools

import jax
import jax.numpy as jnp
from jax import lax
from jax.experimental import pallas as pl
from jax.experimental.pallas import tpu as pltpu
from jax.experimental.pallas import tpu_sc as plsc

_B = 16384
_D = 64
_NW = 32
_BPW = _B // _NW
_CH = 128
_NCH = _BPW // _CH
_W = 8
_L = 16

_mesh = plsc.VectorSubcoreMesh(core_axis_name="c", subcore_axis_name="s")


@functools.partial(
    pl.kernel,
    mesh=_mesh,
    compiler_params=pltpu.CompilerParams(
        use_tc_tiling_on_sc=True, needs_layout_passes=False),
    out_type=jax.ShapeDtypeStruct((_B, _D), jnp.float32),
    scratch_types=[
        pltpu.VMEM((_BPW, _D), jnp.float32),
        pltpu.VMEM((_BPW,), jnp.int32),
        pltpu.SemaphoreType.DMA,
    ],
)
def _gather_z(idx_hbm, emb_hbm, z_out, out_v, idx_v, sem):
    wid = lax.axis_index("s") * 2 + lax.axis_index("c")
    pltpu.sync_copy(idx_hbm.at[pl.ds(wid * _BPW, _BPW)], idx_v)

    def body(g, carry):
        vec = idx_v[pl.ds(g * _L, _L)]
        for k in range(_L):
            pltpu.async_copy(
                emb_hbm.at[vec[k]], out_v.at[g * _L + k], sem)
        @pl.when(g >= 5)
        def _():
            pltpu.make_async_copy(
                emb_hbm.at[pl.ds(0, _L)], out_v.at[pl.ds(0, _L)], sem).wait()
        return carry

    lax.fori_loop(0, _BPW // _L, body, 0)
    for _ in range(5):
        pltpu.make_async_copy(
            emb_hbm.at[pl.ds(0, _L)], out_v.at[pl.ds(0, _L)], sem).wait()
    pltpu.sync_copy(out_v, z_out.at[pl.ds(wid * _BPW, _BPW)])


@functools.partial(
    pl.kernel,
    mesh=_mesh,
    compiler_params=pltpu.CompilerParams(
        use_tc_tiling_on_sc=False, needs_layout_passes=False),
    out_type=[
        jax.ShapeDtypeStruct((_NW * _NCH, _CH), jnp.float32),
        jax.ShapeDtypeStruct((_NW * _NCH, _CH), jnp.float32),
    ],
    scratch_types=[
        pltpu.VMEM((_NCH, _CH), jnp.int32),
        pltpu.VMEM((_NCH, _CH), jnp.int32),
        pltpu.VMEM((_NCH, _CH, _W), jnp.float32),
        pltpu.VMEM((_NCH, _CH, _W), jnp.float32),
        pltpu.VMEM((_NCH, _CH), jnp.float32),
        pltpu.VMEM((_NCH, _CH), jnp.float32),
        pltpu.SemaphoreType.DMA,
    ],
)
def _gather_small(idx_hbm, hi3_hbm, inh_hbm, det_hbm,
                  inh_out, det_out,
                  idx_v, hi3_v, inh_rows, det_rows,
                  inh_sel, det_sel, sem):
    wid = lax.axis_index("s") * 2 + lax.axis_index("c")
    base = wid * _NCH
    pltpu.sync_copy(idx_hbm.at[pl.ds(base, _NCH)], idx_v)
    pltpu.sync_copy(hi3_hbm.at[pl.ds(base, _NCH)], hi3_v)
    copies = []
    for j in range(_NCH):
        copies.append(pltpu.async_copy(inh_hbm.at[hi3_v.at[j]], inh_rows.at[j], sem))
        copies.append(pltpu.async_copy(det_hbm.at[hi3_v.at[j]], det_rows.at[j], sem))
    for c in copies:
        c.wait()
    iotas = [lax.iota(jnp.int32, _L) + (_L * t) for t in range(_CH // _L)]
    for j in range(_NCH):
        for t in range(_CH // _L):
            o = _L * t
            lo = lax.bitwise_and(idx_v[j, pl.ds(o, _L)], 7)
            row = iotas[t]
            inh_sel[j, pl.ds(o, _L)] = plsc.load_gather(
                inh_rows.at[j], [row, lo])
            det_sel[j, pl.ds(o, _L)] = plsc.load_gather(
                det_rows.at[j], [row, lo])
    pltpu.sync_copy(inh_sel, inh_out.at[pl.ds(base, _NCH)])
    pltpu.sync_copy(det_sel, det_out.at[pl.ds(base, _NCH)])


def kernel(piano_model, embedding, inharm_embedding, detune_embedding):
    idx = piano_model.astype(jnp.int32)
    idx2d = idx.reshape(_NW * _NCH, _CH)
    hi3_2d = (idx >> 3).reshape(_NW * _NCH, _CH)
    z2 = _gather_z(idx, embedding)
    inh, det = _gather_small(
        idx2d, hi3_2d,
        inharm_embedding.reshape(-1, _W),
        detune_embedding.reshape(-1, _W))
    return (z2.reshape(_B, 1, _D),
            inh.reshape(_B, 1, 1),
            det.reshape(_B, 1, 1))


# speedup vs baseline: 2.5949x; 1.0210x over previous
import functools

import jax
import jax.numpy as jnp
from jax import lax
from jax.experimental import pallas as pl
from jax.experimental.pallas import tpu as pltpu
from jax.experimental.pallas import tpu_sc as plsc

_B = 16384
_D = 64
_NW = 32
_BPW = _B // _NW
_CH = 128
_NCH = _BPW // _CH
_W = 8
_L = 16

_mesh = plsc.VectorSubcoreMesh(core_axis_name="c", subcore_axis_name="s")


@functools.partial(
    pl.kernel,
    mesh=_mesh,
    compiler_params=pltpu.CompilerParams(
        use_tc_tiling_on_sc=True, needs_layout_passes=False),
    out_type=jax.ShapeDtypeStruct((_B, _D), jnp.float32),
    scratch_types=[
        pltpu.VMEM((_BPW, _D), jnp.float32),
        pltpu.VMEM((_BPW,), jnp.int32),
        pltpu.SemaphoreType.DMA,
    ],
)
def _gather_z(idx_hbm, emb_hbm, z_out, out_v, idx_v, sem):
    wid = lax.axis_index("s") * 2 + lax.axis_index("c")
    pltpu.sync_copy(idx_hbm.at[pl.ds(wid * _BPW, _BPW)], idx_v)

    def body(g, carry):
        vec = idx_v[pl.ds(g * _L, _L)]
        for k in range(_L):
            pltpu.async_copy(
                emb_hbm.at[vec[k]], out_v.at[g * _L + k], sem)
        @pl.when(g >= 11)
        def _():
            pltpu.make_async_copy(
                emb_hbm.at[pl.ds(0, _L)], out_v.at[pl.ds(0, _L)], sem).wait()
        return carry

    lax.fori_loop(0, _BPW // _L, body, 0)
    for _ in range(11):
        pltpu.make_async_copy(
            emb_hbm.at[pl.ds(0, _L)], out_v.at[pl.ds(0, _L)], sem).wait()
    pltpu.sync_copy(out_v, z_out.at[pl.ds(wid * _BPW, _BPW)])


@functools.partial(
    pl.kernel,
    mesh=_mesh,
    compiler_params=pltpu.CompilerParams(
        use_tc_tiling_on_sc=False, needs_layout_passes=False),
    out_type=[
        jax.ShapeDtypeStruct((_NW * _NCH, _CH), jnp.float32),
        jax.ShapeDtypeStruct((_NW * _NCH, _CH), jnp.float32),
    ],
    scratch_types=[
        pltpu.VMEM((_NCH, _CH), jnp.int32),
        pltpu.VMEM((_NCH, _CH), jnp.int32),
        pltpu.VMEM((_NCH, _CH, _W), jnp.float32),
        pltpu.VMEM((_NCH, _CH, _W), jnp.float32),
        pltpu.VMEM((_NCH, _CH), jnp.float32),
        pltpu.VMEM((_NCH, _CH), jnp.float32),
        pltpu.SemaphoreType.DMA,
    ],
)
def _gather_small(idx_hbm, hi3_hbm, inh_hbm, det_hbm,
                  inh_out, det_out,
                  idx_v, hi3_v, inh_rows, det_rows,
                  inh_sel, det_sel, sem):
    wid = lax.axis_index("s") * 2 + lax.axis_index("c")
    base = wid * _NCH
    pltpu.sync_copy(idx_hbm.at[pl.ds(base, _NCH)], idx_v)
    pltpu.sync_copy(hi3_hbm.at[pl.ds(base, _NCH)], hi3_v)
    copies = []
    for j in range(_NCH):
        copies.append(pltpu.async_copy(inh_hbm.at[hi3_v.at[j]], inh_rows.at[j], sem))
        copies.append(pltpu.async_copy(det_hbm.at[hi3_v.at[j]], det_rows.at[j], sem))
    for c in copies:
        c.wait()
    iotas = [lax.iota(jnp.int32, _L) + (_L * t) for t in range(_CH // _L)]
    for j in range(_NCH):
        for t in range(_CH // _L):
            o = _L * t
            lo = lax.bitwise_and(idx_v[j, pl.ds(o, _L)], 7)
            row = iotas[t]
            inh_sel[j, pl.ds(o, _L)] = plsc.load_gather(
                inh_rows.at[j], [row, lo])
            det_sel[j, pl.ds(o, _L)] = plsc.load_gather(
                det_rows.at[j], [row, lo])
    pltpu.sync_copy(inh_sel, inh_out.at[pl.ds(base, _NCH)])
    pltpu.sync_copy(det_sel, det_out.at[pl.ds(base, _NCH)])


def kernel(piano_model, embedding, inharm_embedding, detune_embedding):
    idx = piano_model.astype(jnp.int32)
    idx2d = idx.reshape(_NW * _NCH, _CH)
    hi3_2d = (idx >> 3).reshape(_NW * _NCH, _CH)
    z2 = _gather_z(idx, embedding)
    inh, det = _gather_small(
        idx2d, hi3_2d,
        inharm_embedding.reshape(-1, _W),
        detune_embedding.reshape(-1, _W))
    return (z2.reshape(_B, 1, _D),
            inh.reshape(_B, 1, 1),
            det.reshape(_B, 1, 1))
